# baseline (speedup 1.0000x reference)
import functools

import jax
import jax.numpy as jnp
from jax import lax
from jax.experimental import pallas as pl
from jax.experimental.pallas import tpu as pltpu
from jax.experimental.pallas import tpu_sc as plsc


@functools.lru_cache(maxsize=None)
def _make_gather(Bz, Sz, V, D, DP, NBUF):
    info = plsc.get_sparse_core_info()
    NC, NS = info.num_cores, info.num_subcores
    NW = NC * NS
    assert Bz % NW == 0
    rows_per_w = Bz // NW
    assert rows_per_w % NBUF == 0 and rows_per_w // NBUF >= 2
    n_idx = rows_per_w * Sz
    mesh = plsc.VectorSubcoreMesh(core_axis_name="c", subcore_axis_name="s")

    @functools.partial(
        pl.kernel,
        mesh=mesh,
        out_type=jax.ShapeDtypeStruct((Bz, Sz, DP), jnp.float32),
        scratch_types=[
            pltpu.VMEM((n_idx,), jnp.int32),
            [pltpu.VMEM((Sz, DP), jnp.float32) for _ in range(NBUF)],
            [pltpu.SemaphoreType.DMA for _ in range(NBUF)],
            [pltpu.SemaphoreType.DMA for _ in range(NBUF)],
        ],
    )
    def gather_kernel(idx_hbm, wide_hbm, out_hbm, idx_v, rows, gsem, osem):
        wid = lax.axis_index("s") * NC + lax.axis_index("c")
        base = wid * rows_per_w
        pltpu.sync_copy(idx_hbm.at[pl.ds(base * Sz, n_idx)], idx_v)

        def start_gather(i, b):
            pltpu.async_copy(
                wide_hbm.at[idx_v.at[pl.ds(i * Sz, Sz)]], rows[b], gsem[b]
            )

        def wait_gather(b):
            pltpu.make_async_copy(wide_hbm.at[pl.ds(0, Sz)], rows[b], gsem[b]).wait()

        def start_store(i, b):
            pltpu.async_copy(rows[b], out_hbm.at[base + i], osem[b])

        def wait_store(b):
            pltpu.make_async_copy(wide_hbm.at[pl.ds(0, Sz)], rows[b], osem[b]).wait()

        for b in range(NBUF):
            start_gather(b, b)

        def steady(g, carry):
            for b in range(NBUF):
                i = g * NBUF + b
                wait_gather(b)
                start_store(i, b)
                wait_store(b)
                start_gather(i + NBUF, b)
            return carry

        lax.fori_loop(0, rows_per_w // NBUF - 1, steady, 0)

        for b in range(NBUF):
            i = rows_per_w - NBUF + b
            wait_gather(b)
            start_store(i, b)
        for b in range(NBUF):
            wait_store(b)

    return gather_kernel


def kernel(item_id, item_embeddings):
    Bz, Sz = item_id.shape
    V, D = item_embeddings.shape
    DP = 128
    idx = item_id.reshape(Bz * Sz).astype(jnp.int32)
    table_wide = jnp.pad(item_embeddings, ((0, 0), (0, DP - D)))
    out_wide = _make_gather(Bz, Sz, V, D, DP, 4)(idx, table_wide)
    return out_wide[:, :, :D]
